# baseline (speedup 1.0000x reference)
import functools
import math

import jax
import jax.numpy as jnp
from jax import lax
from jax.experimental import pallas as pl
from jax.experimental.pallas import tpu as pltpu
from jax.experimental.pallas import tpu_sc as plsc

D_H = 64
CHUNK = 128
NUM_CORES = 2
NUM_SUBCORES = 16
NW = NUM_CORES * NUM_SUBCORES


def _emb_body(seq_len, rows_per_w, src_hbm, pe_hbm, table_hbm, out_hbm,
              idx_v, rows_v, pe_v, sem):
    wid = lax.axis_index("s") * NUM_CORES + lax.axis_index("c")
    base = wid * rows_per_w
    n_chunks = rows_per_w // CHUNK
    pltpu.sync_copy(pe_hbm, pe_v)

    def chunk_body(c, carry):
        row0 = base + c * CHUNK
        pltpu.sync_copy(src_hbm.at[pl.ds(row0, CHUNK)], idx_v)
        pltpu.async_copy(table_hbm.at[idx_v], rows_v, sem).wait()
        pos0 = lax.rem(c * CHUNK, seq_len)

        def row_body(r, carry2):
            pr = pos0 + r
            for k in range(D_H // 16):
                v = rows_v[r, pl.ds(k * 16, 16)]
                p = pe_v[pr, pl.ds(k * 16, 16)]
                rows_v[r, pl.ds(k * 16, 16)] = v * 8.0 + p
            return carry2

        lax.fori_loop(0, CHUNK, row_body, 0)
        pltpu.sync_copy(rows_v, out_hbm.at[pl.ds(row0, CHUNK)])
        return carry

    lax.fori_loop(0, n_chunks, chunk_body, 0)


def _build_sc_call(n_rows, seq_len):
    rows_per_w = n_rows // NW
    mesh = plsc.VectorSubcoreMesh(core_axis_name="c", subcore_axis_name="s")
    return functools.partial(
        pl.kernel,
        out_type=jax.ShapeDtypeStruct((n_rows, D_H), jnp.float32),
        mesh=mesh,
        scratch_types=[
            pltpu.VMEM((CHUNK,), jnp.int32),
            pltpu.VMEM((CHUNK, D_H), jnp.float32),
            pltpu.VMEM((seq_len + CHUNK, D_H), jnp.float32),
            pltpu.SemaphoreType.DMA,
        ],
        compiler_params=pltpu.CompilerParams(use_tc_tiling_on_sc=False),
    )(functools.partial(_emb_body, seq_len, rows_per_w))


def kernel(src, table, pe):
    b, l = src.shape
    n_rows = b * l
    assert n_rows % (NW * CHUNK) == 0
    src_flat = src.reshape(n_rows)
    pe_seq = pe[0, :l, :]
    pe_rep = jnp.concatenate([pe_seq, pe_seq[:CHUNK]], axis=0)
    out = _build_sc_call(n_rows, l)(src_flat, pe_rep, table)
    return out.reshape(b, l, D_H)


# speedup vs baseline: 1.4349x; 1.4349x over previous
import functools
import math

import jax
import jax.numpy as jnp
from jax import lax
from jax.experimental import pallas as pl
from jax.experimental.pallas import tpu as pltpu
from jax.experimental.pallas import tpu_sc as plsc

D_H = 64
STREAM = 128
CHUNK = 2 * STREAM
NUM_CORES = 2
NUM_SUBCORES = 16
NW = NUM_CORES * NUM_SUBCORES


def _emb_body(seq_len, rows_per_w, src_hbm, pe_hbm, table_hbm, out_hbm,
              idx_v, rows_v, pe_v, gsem0, gsem1, ssem0, ssem1):
    gsem = (gsem0, gsem1)
    ssem = (ssem0, ssem1)
    wid = lax.axis_index("s") * NUM_CORES + lax.axis_index("c")
    n_chunks = rows_per_w // CHUNK
    blk_base = wid * (rows_per_w // STREAM)

    pltpu.sync_copy(pe_hbm, pe_v)
    pltpu.sync_copy(src_hbm.at[pl.ds(blk_base, rows_per_w // STREAM)], idx_v)

    def fire_gather(c, b):
        for s in range(2):
            pltpu.async_copy(table_hbm.at[idx_v.at[c * 2 + s]],
                             rows_v.at[b, s], gsem[b])

    def wait_gather(b):
        for s in range(2):
            pltpu.make_async_copy(table_hbm.at[idx_v.at[s]],
                                  rows_v.at[b, s], gsem[b]).wait()

    def fire_store(c, b):
        pltpu.async_copy(rows_v.at[b], out_hbm.at[pl.ds(blk_base + c * 2, 2)],
                         ssem[b])

    def wait_store(b):
        pltpu.make_async_copy(rows_v.at[b], out_hbm.at[pl.ds(blk_base, 2)],
                              ssem[b]).wait()

    def chunk_step(c, b, nb):
        @pl.when(c + 1 < n_chunks)
        def _():
            @pl.when(c >= 1)
            def _():
                wait_store(nb)
            fire_gather(c + 1, nb)

        wait_gather(b)
        pos0 = lax.rem(c * CHUNK, seq_len)
        for s in range(2):
            ps = pos0 + s * STREAM

            @pl.loop(0, STREAM, unroll=4)
            def _(r):
                pr = ps + r
                for k in range(D_H // 16):
                    v = rows_v[b, s, r, pl.ds(16 * k, 16)]
                    p = pe_v[pr, pl.ds(16 * k, 16)]
                    rows_v[b, s, r, pl.ds(16 * k, 16)] = v * 8.0 + p

        fire_store(c, b)

    fire_gather(0, 0)

    @pl.loop(0, n_chunks // 2)
    def _(c2):
        chunk_step(c2 * 2, 0, 1)
        chunk_step(c2 * 2 + 1, 1, 0)

    wait_store(0)
    wait_store(1)


def _build_sc_call(n_rows, seq_len):
    rows_per_w = n_rows // NW
    mesh = plsc.VectorSubcoreMesh(core_axis_name="c", subcore_axis_name="s")
    return functools.partial(
        pl.kernel,
        out_type=jax.ShapeDtypeStruct((n_rows // STREAM, STREAM, D_H),
                                      jnp.float32),
        mesh=mesh,
        scratch_types=[
            pltpu.VMEM((rows_per_w // STREAM, STREAM), jnp.int32),
            pltpu.VMEM((2, 2, STREAM, D_H), jnp.float32),
            pltpu.VMEM((seq_len + CHUNK, D_H), jnp.float32),
            pltpu.SemaphoreType.DMA,
            pltpu.SemaphoreType.DMA,
            pltpu.SemaphoreType.DMA,
            pltpu.SemaphoreType.DMA,
        ],
        compiler_params=pltpu.CompilerParams(use_tc_tiling_on_sc=False),
    )(functools.partial(_emb_body, seq_len, rows_per_w))


def kernel(src, table, pe):
    b, l = src.shape
    n_rows = b * l
    assert n_rows % (NW * CHUNK) == 0
    src_blk = src.reshape(n_rows // STREAM, STREAM)
    pe_seq = pe[0, :l, :]
    pe_rep = jnp.concatenate([pe_seq, pe_seq, pe_seq[:CHUNK - l]], axis=0)
    out = _build_sc_call(n_rows, l)(src_blk, pe_rep, table)
    return out.reshape(b, l, D_H)
